# baseline (speedup 1.0000x reference)
import jax
import jax.numpy as jnp
from jax.experimental import pallas as pl
from jax.experimental.pallas import tpu as pltpu


def _vq_block(z_ref, w_ref, zsq_ref, q_ref, zq_ref,
              w2_ref, w3_ref, wsq_ref, iotaf_ref):
    kk = w_ref.shape[0]

    @pl.when(pl.program_id(0) == 0)
    def _init():
        w = w_ref[...]
        w2_ref[...] = w + w
        w_hi = w.astype(jnp.bfloat16).astype(jnp.float32)
        r1 = w - w_hi
        w_mid = r1.astype(jnp.bfloat16).astype(jnp.float32)
        w_lo = r1 - w_mid
        w3_ref[...] = jnp.concatenate([w_hi, w_mid, w_lo], axis=1)
        wsq_ref[...] = jnp.sum(w * w, axis=1)[:, None]
        iotaf_ref[...] = jax.lax.broadcasted_iota(
            jnp.int32, iotaf_ref.shape, 0).astype(jnp.float32)

    zt = z_ref[0]
    zsq = zsq_ref[0]
    zw2 = jax.lax.dot_general(w2_ref[...], zt, (((1,), (0,)), ((), ())),
                              preferred_element_type=jnp.float32)
    dist = zsq - zw2 + wsq_ref[...]
    minv = jnp.min(dist, axis=0, keepdims=True)
    iotaf = iotaf_ref[...]
    qf = jnp.min(jnp.where(dist == minv, iotaf, float(kk)), axis=0)
    q_ref[0, 0, :] = qf.astype(jnp.int32)
    onehot = (iotaf == qf[None, :]).astype(jnp.float32)
    zq3 = jax.lax.dot_general(w3_ref[...], onehot, (((0,), (0,)), ((), ())),
                              preferred_element_type=jnp.float32)
    dd = zt.shape[0]
    zq_ref[0] = (zq3[:dd] + zq3[dd:2 * dd]) + zq3[2 * dd:]


def _vq_call(z3, weights, zsq3):
    N, D, BT = z3.shape
    K = weights.shape[0]
    q3, zq = pl.pallas_call(
        _vq_block,
        grid=(N,),
        in_specs=[
            pl.BlockSpec((1, D, BT), lambda i: (i, 0, 0)),
            pl.BlockSpec((K, D), lambda i: (0, 0)),
            pl.BlockSpec((1, 1, BT), lambda i: (i, 0, 0)),
        ],
        out_specs=[
            pl.BlockSpec((1, 1, BT), lambda i: (i, 0, 0)),
            pl.BlockSpec((1, D, BT), lambda i: (i, 0, 0)),
        ],
        out_shape=[
            jax.ShapeDtypeStruct((N, 1, BT), jnp.int32),
            jax.ShapeDtypeStruct((N, D, BT), jnp.float32),
        ],
        scratch_shapes=[
            pltpu.VMEM((K, D), jnp.float32),
            pltpu.VMEM((K, 3 * D), jnp.float32),
            pltpu.VMEM((K, 1), jnp.float32),
            pltpu.VMEM((K, BT), jnp.float32),
        ],
    )(z3, weights, zsq3)
    return q3, zq


def kernel(z_e, weights):
    N, D, H, W = z_e.shape
    BT = H * W
    z3 = z_e.reshape(N, D, BT)
    zsq3 = (z_e ** 2).sum(axis=1).reshape(N, 1, BT)
    devs = jax.devices()
    nd = len(devs)
    while nd > 1 and N % nd != 0:
        nd -= 1
    if nd > 1:
        mesh = jax.sharding.Mesh(devs[:nd], ("b",))
        P = jax.sharding.PartitionSpec
        fn = jax.shard_map(
            _vq_call, mesh=mesh,
            in_specs=(P("b", None, None), P(None, None), P("b", None, None)),
            out_specs=(P("b", None, None), P("b", None, None)),
            check_vma=False,
        )
        q3, zq = fn(z3, weights, zsq3)
    else:
        q3, zq = _vq_call(z3, weights, zsq3)
    q = q3.reshape(N, H, W)
    z_q = zq.reshape(N, D, H, W)
    return q, z_q


# speedup vs baseline: 10.0541x; 10.0541x over previous
import functools

import jax
import jax.numpy as jnp
from jax import lax
from jax.experimental import pallas as pl
from jax.experimental.pallas import tpu as pltpu
from jax.experimental.pallas import tpu_sc as plsc


def _vq_block(z_ref, w_ref, zsq_ref, q_ref, w2_ref, wsq_ref, iotaf_ref):
    kk = w_ref.shape[0]

    @pl.when(pl.program_id(0) == 0)
    def _init():
        w = w_ref[...]
        w2_ref[...] = w + w
        wsq_ref[...] = jnp.sum(w * w, axis=1)[:, None]
        iotaf_ref[...] = jax.lax.broadcasted_iota(
            jnp.int32, iotaf_ref.shape, 0).astype(jnp.float32)

    zt = z_ref[0]
    zsq = zsq_ref[0]
    zw2 = jax.lax.dot_general(w2_ref[...], zt, (((1,), (0,)), ((), ())),
                              preferred_element_type=jnp.float32)
    dist = zsq - zw2 + wsq_ref[...]
    minv = jnp.min(dist, axis=0, keepdims=True)
    iotaf = iotaf_ref[...]
    qf = jnp.min(jnp.where(dist == minv, iotaf, float(kk)), axis=0)
    q_ref[0, 0, :] = qf.astype(jnp.int32)


def _make_sc_gather(V, D, B):
    info = plsc.get_sparse_core_info()
    NC, NS = info.num_cores, info.num_subcores
    NW = NC * NS
    assert B % (8 * NW) == 0
    b_per_w = B // NW
    mesh = plsc.VectorSubcoreMesh(core_axis_name="c", subcore_axis_name="s")

    @functools.partial(
        pl.kernel, mesh=mesh,
        out_type=jax.ShapeDtypeStruct((B, D), jnp.float32),
        scratch_types=[
            pltpu.VMEM((b_per_w,), jnp.int32),
            pltpu.VMEM((b_per_w, D), jnp.float32),
            pltpu.SemaphoreType.DMA,
        ],
    )
    def k(table_hbm, idx_hbm, out_hbm, idx_v, rows_v, sem):
        wid = lax.axis_index("s") * NC + lax.axis_index("c")
        base = wid * b_per_w
        pltpu.sync_copy(idx_hbm.at[pl.ds(base, b_per_w)], idx_v)
        pltpu.async_copy(table_hbm.at[idx_v], rows_v, sem).wait()
        pltpu.sync_copy(rows_v, out_hbm.at[pl.ds(base, b_per_w)])

    return k


def kernel(z_e, weights):
    N, D, H, W = z_e.shape
    K = weights.shape[0]
    BT = H * W
    z3 = z_e.reshape(N, D, BT)
    zsq3 = (z_e ** 2).sum(axis=1).reshape(N, 1, BT)
    q3 = pl.pallas_call(
        _vq_block,
        grid=(N,),
        in_specs=[
            pl.BlockSpec((1, D, BT), lambda i: (i, 0, 0)),
            pl.BlockSpec((K, D), lambda i: (0, 0)),
            pl.BlockSpec((1, 1, BT), lambda i: (i, 0, 0)),
        ],
        out_specs=pl.BlockSpec((1, 1, BT), lambda i: (i, 0, 0)),
        out_shape=jax.ShapeDtypeStruct((N, 1, BT), jnp.int32),
        scratch_shapes=[
            pltpu.VMEM((K, D), jnp.float32),
            pltpu.VMEM((K, 1), jnp.float32),
            pltpu.VMEM((K, BT), jnp.float32),
        ],
    )(z3, weights, zsq3)
    qflat = q3.reshape(N * BT)
    wpad = jnp.pad(weights, ((0, 0), (0, 128 - D)))
    zq_tok = _make_sc_gather(K, 128, N * BT)(wpad, qflat)
    z_q = (zq_tok[:, :D].reshape(N, BT, D)
           .transpose(0, 2, 1).reshape(N, D, H, W))
    q = q3.reshape(N, H, W)
    return q, z_q


# speedup vs baseline: 14.8164x; 1.4737x over previous
import jax
import jax.numpy as jnp
from jax.experimental import pallas as pl
from jax.experimental.pallas import tpu as pltpu


def _vq_block(z_ref, w_ref, zsq_ref, q_ref, zq_ref,
              w2_ref, w3_ref, wsq_ref, iotaf_ref):
    kk = w_ref.shape[0]

    @pl.when(pl.program_id(0) == 0)
    def _init():
        w = w_ref[...]
        w2_ref[...] = w + w
        w_hi = w.astype(jnp.bfloat16).astype(jnp.float32)
        r1 = w - w_hi
        w_mid = r1.astype(jnp.bfloat16).astype(jnp.float32)
        w_lo = r1 - w_mid
        w3_ref[...] = jnp.concatenate([w_hi, w_mid, w_lo], axis=1)
        wsq_ref[...] = jnp.sum(w * w, axis=1)[:, None]
        iotaf_ref[...] = jax.lax.broadcasted_iota(
            jnp.int32, iotaf_ref.shape, 0).astype(jnp.float32)

    zt = z_ref[0]
    zsq = zsq_ref[0]
    zw2 = jax.lax.dot_general(w2_ref[...], zt, (((1,), (0,)), ((), ())),
                              preferred_element_type=jnp.float32)
    dist = zsq - zw2 + wsq_ref[...]
    minv = jnp.min(dist, axis=0, keepdims=True)
    iotaf = iotaf_ref[...]
    qf = jnp.min(jnp.where(dist == minv, iotaf, float(kk)), axis=0)
    q_ref[0, 0, :] = qf.astype(jnp.int32)
    onehot = (iotaf == qf[None, :]).astype(jnp.float32)
    zq3 = jax.lax.dot_general(w3_ref[...], onehot, (((0,), (0,)), ((), ())),
                              preferred_element_type=jnp.float32)
    dd = zt.shape[0]
    zq_ref[0] = (zq3[:dd] + zq3[dd:2 * dd]) + zq3[2 * dd:]


def kernel(z_e, weights):
    N, D, H, W = z_e.shape
    K = weights.shape[0]
    BT = H * W
    z3 = z_e.reshape(N, D, BT)
    zsq3 = (z_e ** 2).sum(axis=1).reshape(N, 1, BT)
    q3, zq = pl.pallas_call(
        _vq_block,
        grid=(N,),
        in_specs=[
            pl.BlockSpec((1, D, BT), lambda i: (i, 0, 0)),
            pl.BlockSpec((K, D), lambda i: (0, 0)),
            pl.BlockSpec((1, 1, BT), lambda i: (i, 0, 0)),
        ],
        out_specs=[
            pl.BlockSpec((1, 1, BT), lambda i: (i, 0, 0)),
            pl.BlockSpec((1, D, BT), lambda i: (i, 0, 0)),
        ],
        out_shape=[
            jax.ShapeDtypeStruct((N, 1, BT), jnp.int32),
            jax.ShapeDtypeStruct((N, D, BT), jnp.float32),
        ],
        scratch_shapes=[
            pltpu.VMEM((K, D), jnp.float32),
            pltpu.VMEM((K, 3 * D), jnp.float32),
            pltpu.VMEM((K, 1), jnp.float32),
            pltpu.VMEM((K, BT), jnp.float32),
        ],
    )(z3, weights, zsq3)
    q = q3.reshape(N, H, W)
    z_q = zq.reshape(N, D, H, W)
    return q, z_q
